# baseline (speedup 1.0000x reference)
import jax
import jax.numpy as jnp
from jax import lax
from jax.experimental import pallas as pl
from jax.experimental.pallas import tpu as pltpu
from jax.experimental.pallas import tpu_sc as plsc

_MAXLEN = 8192
_EMBED = 128
_BATCH = 4
_TC_BATCH = 2
_SC_BATCH = _BATCH - _TC_BATCH
_TBLK = 1024
_NC = 2
_NS = 16
_ROWS = _MAXLEN // (_NC * _NS)
_LANES = 16


def _tc_body(x_ref, p_ref, o_ref):
    o_ref[...] = x_ref[...] + p_ref[...]


def _tc_half(x, pos_table):
    return pl.pallas_call(
        _tc_body,
        grid=(_TC_BATCH, _MAXLEN // _TBLK),
        in_specs=[
            pl.BlockSpec((1, _TBLK, _EMBED), lambda b, t: (b, t, 0)),
            pl.BlockSpec((_TBLK, _EMBED), lambda b, t: (t, 0)),
        ],
        out_specs=pl.BlockSpec((1, _TBLK, _EMBED), lambda b, t: (b, t, 0)),
        out_shape=jax.ShapeDtypeStruct((_TC_BATCH, _MAXLEN, _EMBED),
                                       jnp.float32),
    )(x, pos_table)


def _add_rows(buf, pos_v):
    @plsc.parallel_loop(0, _ROWS, step=1)
    def _row(r):
        for c in range(_EMBED // _LANES):
            sl = pl.ds(c * _LANES, _LANES)
            buf[r, sl] = buf[r, sl] + pos_v[r, sl]


def _sc_body(x_hbm, pos_hbm, out_hbm, pos_v, buf0, buf1,
             lsem0, lsem1, ssem0, ssem1):
    wid = lax.axis_index("s") * _NC + lax.axis_index("c")
    tsl = pl.ds(wid * _ROWS, _ROWS)

    bufs = (buf0, buf1)
    lsems = (lsem0, lsem1)
    ssems = (ssem0, ssem1)

    loads = {0: pltpu.async_copy(x_hbm.at[_TC_BATCH, tsl], buf0, lsem0)}
    pltpu.sync_copy(pos_hbm.at[tsl], pos_v)

    stores = {}
    for b in range(_SC_BATCH):
        i = b & 1
        loads[b].wait()
        if b + 1 < _SC_BATCH:
            if b >= 1:
                stores[b - 1].wait()
            loads[b + 1] = pltpu.async_copy(
                x_hbm.at[_TC_BATCH + b + 1, tsl], bufs[1 - i], lsems[1 - i])
        _add_rows(bufs[i], pos_v)
        stores[b] = pltpu.async_copy(bufs[i], out_hbm.at[b, tsl], ssems[i])
    for b in range(max(0, _SC_BATCH - 2), _SC_BATCH):
        stores[b].wait()


def _sc_half(x, pos_table):
    mesh = plsc.VectorSubcoreMesh(core_axis_name="c", subcore_axis_name="s",
                                  num_cores=_NC, num_subcores=_NS)
    run = pl.kernel(
        _sc_body,
        out_type=jax.ShapeDtypeStruct((_SC_BATCH, _MAXLEN, _EMBED),
                                      jnp.float32),
        mesh=mesh,
        scratch_types=[
            pltpu.VMEM((_ROWS, _EMBED), jnp.float32),
            pltpu.VMEM((_ROWS, _EMBED), jnp.float32),
            pltpu.VMEM((_ROWS, _EMBED), jnp.float32),
            pltpu.SemaphoreType.DMA,
            pltpu.SemaphoreType.DMA,
            pltpu.SemaphoreType.DMA,
            pltpu.SemaphoreType.DMA,
        ],
    )
    return run(x, pos_table)


def kernel(x, pos_table):
    out_tc = _tc_half(x, pos_table)
    out_sc = _sc_half(x, pos_table)
    return jnp.concatenate([out_tc, out_sc], axis=0)


# speedup vs baseline: 1.7865x; 1.7865x over previous
import jax
import jax.numpy as jnp
from jax.experimental import pallas as pl

_MAXLEN = 8192
_EMBED = 128
_BATCH = 4
_TBLK = 1024


def _add_body(x_ref, p_ref, o_ref):
    o_ref[...] = x_ref[...] + p_ref[...]


def kernel(x, pos_table):
    grid = (_MAXLEN // _TBLK, _BATCH)
    return pl.pallas_call(
        _add_body,
        grid=grid,
        in_specs=[
            pl.BlockSpec((1, _TBLK, _EMBED), lambda t, b: (b, t, 0)),
            pl.BlockSpec((_TBLK, _EMBED), lambda t, b: (t, 0)),
        ],
        out_specs=pl.BlockSpec((1, _TBLK, _EMBED), lambda t, b: (b, t, 0)),
        out_shape=jax.ShapeDtypeStruct((_BATCH, _MAXLEN, _EMBED), jnp.float32),
    )(x, pos_table)


# speedup vs baseline: 2.4407x; 1.3662x over previous
import jax
import jax.numpy as jnp
from jax.experimental import pallas as pl

_MAXLEN = 8192
_EMBED = 128
_BATCH = 4
_TBLK = 2048


def _add_body(x_ref, p_ref, o_ref):
    o_ref[...] = x_ref[...] + p_ref[...]


def kernel(x, pos_table):
    grid = (_MAXLEN // _TBLK, _BATCH)
    return pl.pallas_call(
        _add_body,
        grid=grid,
        in_specs=[
            pl.BlockSpec((1, _TBLK, _EMBED), lambda t, b: (b, t, 0)),
            pl.BlockSpec((_TBLK, _EMBED), lambda t, b: (t, 0)),
        ],
        out_specs=pl.BlockSpec((1, _TBLK, _EMBED), lambda t, b: (b, t, 0)),
        out_shape=jax.ShapeDtypeStruct((_BATCH, _MAXLEN, _EMBED), jnp.float32),
    )(x, pos_table)


# speedup vs baseline: 3.1003x; 1.2703x over previous
import jax
import jax.numpy as jnp
from jax.experimental import pallas as pl

_MAXLEN = 8192
_EMBED = 128
_BATCH = 4
_TBLK = 4096


def _add_body(x_ref, p_ref, o_ref):
    o_ref[...] = x_ref[...] + p_ref[...]


def kernel(x, pos_table):
    grid = (_MAXLEN // _TBLK, _BATCH)
    return pl.pallas_call(
        _add_body,
        grid=grid,
        in_specs=[
            pl.BlockSpec((1, _TBLK, _EMBED), lambda t, b: (b, t, 0)),
            pl.BlockSpec((_TBLK, _EMBED), lambda t, b: (t, 0)),
        ],
        out_specs=pl.BlockSpec((1, _TBLK, _EMBED), lambda t, b: (b, t, 0)),
        out_shape=jax.ShapeDtypeStruct((_BATCH, _MAXLEN, _EMBED), jnp.float32),
    )(x, pos_table)


# speedup vs baseline: 3.4225x; 1.1039x over previous
import jax
import jax.numpy as jnp
from jax.experimental import pallas as pl

_MAXLEN = 8192
_EMBED = 128
_BATCH = 4
_TBLK = 8192


def _add_body(x_ref, p_ref, o_ref):
    o_ref[...] = x_ref[...] + p_ref[...]


def kernel(x, pos_table):
    grid = (_MAXLEN // _TBLK, _BATCH)
    return pl.pallas_call(
        _add_body,
        grid=grid,
        in_specs=[
            pl.BlockSpec((1, _TBLK, _EMBED), lambda t, b: (b, t, 0)),
            pl.BlockSpec((_TBLK, _EMBED), lambda t, b: (t, 0)),
        ],
        out_specs=pl.BlockSpec((1, _TBLK, _EMBED), lambda t, b: (b, t, 0)),
        out_shape=jax.ShapeDtypeStruct((_BATCH, _MAXLEN, _EMBED), jnp.float32),
    )(x, pos_table)
